# Initial kernel scaffold; baseline (speedup 1.0000x reference)
import jax
import jax.numpy as jnp
from jax.experimental import pallas as pl


def kernel(fine_scale_h_d, prolongation_map_fine_to_coarse):
    raise NotImplementedError("write your pallas kernel here")


# speedup vs baseline: 3.8041x; 3.8041x over previous
import functools

import jax
import jax.numpy as jnp
import numpy as np
from jax import lax
from jax.experimental import pallas as pl
from jax.experimental.pallas import tpu as pltpu
from jax.experimental.pallas import tpu_sc as plsc

N_ROWS = 320000
D = 128
N_SEG = 50000
N_PASS = 4
SEGCHUNK = 6256
G = 64
RID_BITS = 19


def _make_kernel(n_rows=N_ROWS, d=D, n_seg=N_SEG, n_pass=N_PASS,
                 segchunk=SEGCHUNK, g=G, rid_bits=RID_BITS, interpret=False):
    n_cores, n_sub = 2, 16
    n_regions = n_cores * n_pass
    last_len = n_seg - (n_regions - 1) * segchunk
    assert 0 < last_len <= segchunk and last_len % 16 == 0
    assert segchunk % 16 == 0 and n_rows % n_sub == 0
    rows_per_tile = n_rows // n_sub
    assert rows_per_tile % 16 == 0
    cap = ((rows_per_tile + g - 1) // g) * g + g
    trash = segchunk
    acc_rows = trash + 16
    ngroups = segchunk // 16
    ngroups_last = last_len // 16
    k_iter = (ngroups + n_sub - 1) // n_sub
    vec_iters = rows_per_tile // 16
    histpad = ((segchunk + 127) // 128) * 128
    hist_iters = histpad // 16
    rid_mask = (1 << rid_bits) - 1
    gsh = g.bit_length() - 1
    assert n_rows <= (1 << rid_bits) and (trash << rid_bits) < (1 << 32)
    trash_word = np.uint32(trash << rid_bits).astype(np.int32)

    def _body(fine_hbm, seg_hbm, out_hbm,
              segbuf, arena, rowbuf, loc_stage, rid_stage, hist, tmphist,
              zerobuf, sumbuf, resbuf, acc, hist_all, sem):
        c = lax.axis_index("c")
        s = lax.axis_index("s")
        iota16 = lax.iota(jnp.int32, 16)
        fzero = jnp.zeros((16,), jnp.float32)
        fone = jnp.ones((16,), jnp.float32)
        trash_packed = jnp.full((16,), trash_word, jnp.int32)

        for r in range(16):
            for j in range(d // 16):
                zerobuf[r, pl.ds(16 * j, 16)] = fzero

        row_base = s * rows_per_tile
        pltpu.sync_copy(seg_hbm.at[pl.ds(row_base, rows_per_tile)], segbuf)

        ridbase = row_base + iota16

        for p in range(n_pass):
            q = c * n_pass + p
            bp = q * segchunk
            ngr = jnp.where(q == n_regions - 1, ngroups_last, ngroups)

            def hzero_body(i, _):
                plsc.store_scatter(hist, [16 * i + iota16], fzero)
                return 0

            lax.fori_loop(0, hist_iters, hzero_body, 0)

            def comp_body(i, off):
                seg = segbuf[pl.ds(16 * i, 16)]
                loc = seg - bp
                m = (loc >= 0) & (loc < segchunk)
                locc = jnp.where(m, loc, 0)
                cum = plsc.cumsum(m.astype(jnp.int32))
                pos = jnp.where(m, off + cum - 1, cap - 16)
                packed = (ridbase + 16 * i) | (locc << rid_bits)
                plsc.store_scatter(arena, [pos], packed, mask=m)
                plsc.addupdate_scatter(hist, [locc], fone, mask=m)
                return off + jnp.max(cum)

            off = lax.fori_loop(0, vec_iters, comp_body, jnp.int32(0))

            pltpu.sync_copy(hist, hist_all.at[s])

            nch = lax.shift_right_logical(off + (g - 1), gsh)
            fl = jnp.bitwise_and(off, -16)
            keep = iota16 < (off - fl)
            plsc.store_scatter(arena, [fl + iota16], trash_packed,
                               mask=jnp.logical_not(keep))
            n_pad = jnp.maximum(
                lax.shift_right_arithmetic(nch * g - fl - 16, 4), 0)
            for k in range(g // 16 - 1):
                @pl.when(k < n_pad)
                def _():
                    plsc.store_scatter(arena, [fl + 16 + 16 * k + iota16],
                                       trash_packed)

            def zero_body(k, _):
                gi = s + n_sub * k

                @pl.when(gi < ngr)
                def _():
                    r0 = pl.multiple_of(16 * gi, 16)
                    pltpu.sync_copy(zerobuf, acc.at[pl.ds(r0, 16)])
                return 0

            lax.fori_loop(0, k_iter, zero_body, 0)
            plsc.subcore_barrier()

            def acc_body(gi, _):
                gg = gi * g
                for j in range(g // 16):
                    v = arena[pl.ds(gg + 16 * j, 16)]
                    rid_stage[pl.ds(16 * j, 16)] = v & rid_mask
                    loc_stage[0, pl.ds(16 * j, 16)] = (
                        lax.shift_right_logical(v, rid_bits))
                pltpu.async_copy(fine_hbm.at[rid_stage], rowbuf, sem).wait()
                pltpu.sync_copy(rowbuf, acc.at[loc_stage.at[0]], add=True)
                return 0

            lax.fori_loop(0, nch, acc_body, 0)
            plsc.subcore_barrier()

            def hzero2_body(k, _):
                gi = s + n_sub * k

                @pl.when(gi < ngr)
                def _():
                    plsc.store_scatter(hist, [16 * gi + iota16], fzero)
                return 0

            lax.fori_loop(0, k_iter, hzero2_body, 0)

            def hsum_t(t, _):
                pltpu.sync_copy(hist_all.at[t], tmphist)

                def hsum_g(k, _):
                    gi = s + n_sub * k

                    @pl.when(gi < ngr)
                    def _():
                        r0 = 16 * gi
                        cur = hist[pl.ds(r0, 16)]
                        plsc.store_scatter(
                            hist, [r0 + iota16],
                            cur + tmphist[pl.ds(r0, 16)])
                    return 0

                lax.fori_loop(0, k_iter, hsum_g, 0)
                return 0

            lax.fori_loop(0, n_sub, hsum_t, 0)

            def div_body(k, _):
                gi = s + n_sub * k

                @pl.when(gi < ngr)
                def _():
                    r0 = pl.multiple_of(16 * gi, 16)
                    pltpu.sync_copy(acc.at[pl.ds(r0, 16)], sumbuf)
                    cvec = hist[pl.ds(r0, 16)]
                    inv = jnp.where(cvec > 0, 1.0 / cvec, 0.0)
                    for r in range(16):
                        ir = inv[r]
                        for j in range(d // 16):
                            resbuf[r, pl.ds(16 * j, 16)] = (
                                sumbuf[r, pl.ds(16 * j, 16)] * ir)
                    row_out = pl.multiple_of(bp + r0, 16)
                    pltpu.sync_copy(resbuf, out_hbm.at[pl.ds(row_out, 16)])
                return 0

            lax.fori_loop(0, k_iter, div_body, 0)
            plsc.subcore_barrier()

    return functools.partial(
        pl.kernel,
        out_type=jax.ShapeDtypeStruct((n_seg, d), jnp.float32),
        compiler_params=pltpu.CompilerParams(needs_layout_passes=False),
        mesh=plsc.VectorSubcoreMesh(core_axis_name="c", subcore_axis_name="s"),
        interpret=interpret,
        scratch_types=[
            pltpu.VMEM((rows_per_tile,), jnp.int32),
            pltpu.VMEM((cap,), jnp.int32),
            pltpu.VMEM((g, d), jnp.float32),
            pltpu.VMEM((1, g), jnp.int32),
            pltpu.VMEM((g,), jnp.int32),
            pltpu.VMEM((histpad,), jnp.float32),
            pltpu.VMEM((histpad,), jnp.float32),
            pltpu.VMEM((16, d), jnp.float32),
            pltpu.VMEM((16, d), jnp.float32),
            pltpu.VMEM((16, d), jnp.float32),
            pltpu.VMEM_SHARED((acc_rows, d), jnp.float32),
            pltpu.VMEM_SHARED((n_sub, histpad), jnp.float32),
            pltpu.SemaphoreType.DMA,
        ],
    )(_body)


_mean_pool_sc = _make_kernel()


def kernel(fine_scale_h_d, prolongation_map_fine_to_coarse):
    return _mean_pool_sc(fine_scale_h_d, prolongation_map_fine_to_coarse)


# speedup vs baseline: 4.2497x; 1.1171x over previous
import functools

import jax
import jax.numpy as jnp
import numpy as np
from jax import lax
from jax.experimental import pallas as pl
from jax.experimental.pallas import tpu as pltpu
from jax.experimental.pallas import tpu_sc as plsc

N_ROWS = 320000
D = 128
N_SEG = 50000
N_PASS = 4
SEGCHUNK = 6256
G = 64
RID_BITS = 19


def _make_kernel(n_rows=N_ROWS, d=D, n_seg=N_SEG, n_pass=N_PASS,
                 segchunk=SEGCHUNK, g=G, rid_bits=RID_BITS, interpret=False):
    n_cores, n_sub = 2, 16
    n_regions = n_cores * n_pass
    last_len = n_seg - (n_regions - 1) * segchunk
    assert 0 < last_len <= segchunk and last_len % 16 == 0
    assert segchunk % 16 == 0 and n_rows % n_sub == 0
    rows_per_tile = n_rows // n_sub
    assert rows_per_tile % 16 == 0
    cap = ((rows_per_tile + g - 1) // g) * g + g
    trash = segchunk
    acc_rows = trash + 16
    ngroups = segchunk // 16
    ngroups_last = last_len // 16
    k_iter = (ngroups + n_sub - 1) // n_sub
    vec_iters = rows_per_tile // 16
    histpad = ((segchunk + 127) // 128) * 128
    hist_iters = histpad // 16
    rid_mask = (1 << rid_bits) - 1
    gsh = g.bit_length() - 1
    assert n_rows <= (1 << rid_bits) and (trash << rid_bits) < (1 << 32)
    trash_word = np.uint32(trash << rid_bits).astype(np.int32)

    def _body(fine_hbm, seg_hbm, out_hbm,
              segbuf, arena, rowbuf_a, rowbuf_b, loc_stage, rid_a, rid_b,
              hist, tmphist, sumbuf, resbuf, acc, hist_all,
              gsem_a, gsem_b, ssem_a, ssem_b):
        c = lax.axis_index("c")
        s = lax.axis_index("s")
        iota16 = lax.iota(jnp.int32, 16)
        fzero = jnp.zeros((16,), jnp.float32)
        fone = jnp.ones((16,), jnp.float32)
        trash_packed = jnp.full((16,), trash_word, jnp.int32)


        row_base = s * rows_per_tile
        pltpu.sync_copy(seg_hbm.at[pl.ds(row_base, rows_per_tile)], segbuf)

        ridbase = row_base + iota16

        for p in range(n_pass):
            q = c * n_pass + p
            bp = q * segchunk
            ngr = jnp.where(q == n_regions - 1, ngroups_last, ngroups)

            for r in range(16):
                for j in range(d // 16):
                    resbuf[r, pl.ds(16 * j, 16)] = fzero

            def hzero_body(i, _):
                plsc.store_scatter(hist, [16 * i + iota16], fzero)
                return 0

            lax.fori_loop(0, hist_iters, hzero_body, 0)

            def comp_body(i, off):
                seg = segbuf[pl.ds(16 * i, 16)]
                loc = seg - bp
                m = (loc >= 0) & (loc < segchunk)
                locc = jnp.where(m, loc, 0)
                cum = plsc.cumsum(m.astype(jnp.int32))
                pos = jnp.where(m, off + cum - 1, cap - 16)
                packed = (ridbase + 16 * i) | (locc << rid_bits)
                plsc.store_scatter(arena, [pos], packed, mask=m)
                plsc.addupdate_scatter(hist, [locc], fone, mask=m)
                return off + jnp.max(cum)

            off = lax.fori_loop(0, vec_iters, comp_body, jnp.int32(0))

            pltpu.sync_copy(hist, hist_all.at[s])

            nch = lax.shift_right_logical(off + (g - 1), gsh)
            fl = jnp.bitwise_and(off, -16)
            keep = iota16 < (off - fl)
            plsc.store_scatter(arena, [fl + iota16], trash_packed,
                               mask=jnp.logical_not(keep))
            n_pad = jnp.maximum(
                lax.shift_right_arithmetic(nch * g - fl - 16, 4), 0)
            for k in range(g // 16 - 1):
                @pl.when(k < n_pad)
                def _():
                    plsc.store_scatter(arena, [fl + 16 + 16 * k + iota16],
                                       trash_packed)

            def zero_body(k, _):
                gi = s + n_sub * k

                @pl.when(gi < ngr)
                def _():
                    r0 = pl.multiple_of(16 * gi, 16)
                    pltpu.sync_copy(resbuf, acc.at[pl.ds(r0, 16)])
                return 0

            lax.fori_loop(0, k_iter, zero_body, 0)
            plsc.subcore_barrier()

            def unpack(gi, rid_stage, slot):
                gg = gi * g
                for j in range(g // 16):
                    v = arena[pl.ds(gg + 16 * j, 16)]
                    rid_stage[pl.ds(16 * j, 16)] = v & rid_mask
                    loc_stage[slot, pl.ds(16 * j, 16)] = (
                        lax.shift_right_logical(v, rid_bits))

            def acc_body(it, _):
                ga = 2 * it
                gb = 2 * it + 1
                b_ok = gb < nch
                unpack(ga, rid_a, 0)
                cp_ga = pltpu.async_copy(fine_hbm.at[rid_a], rowbuf_a,
                                         gsem_a)

                @pl.when(b_ok)
                def _():
                    unpack(gb, rid_b, 1)
                    pltpu.async_copy(fine_hbm.at[rid_b], rowbuf_b, gsem_b)

                cp_ga.wait()
                cp_sa = pltpu.async_copy(rowbuf_a, acc.at[loc_stage.at[0]],
                                         ssem_a, add=True)

                @pl.when(b_ok)
                def _():
                    pltpu.make_async_copy(fine_hbm.at[rid_b], rowbuf_b,
                                          gsem_b).wait()
                    pltpu.async_copy(rowbuf_b, acc.at[loc_stage.at[1]],
                                     ssem_b, add=True)

                cp_sa.wait()

                @pl.when(b_ok)
                def _():
                    pltpu.make_async_copy(rowbuf_b,
                                          acc.at[loc_stage.at[1]],
                                          ssem_b).wait()
                return 0

            n_it = lax.shift_right_logical(nch + 1, 1)
            lax.fori_loop(0, n_it, acc_body, 0)
            plsc.subcore_barrier()

            def hzero2_body(k, _):
                gi = s + n_sub * k

                @pl.when(gi < ngr)
                def _():
                    plsc.store_scatter(hist, [16 * gi + iota16], fzero)
                return 0

            lax.fori_loop(0, k_iter, hzero2_body, 0)

            def hsum_t(t, _):
                pltpu.sync_copy(hist_all.at[t], tmphist)

                def hsum_g(k, _):
                    gi = s + n_sub * k

                    @pl.when(gi < ngr)
                    def _():
                        r0 = 16 * gi
                        cur = hist[pl.ds(r0, 16)]
                        plsc.store_scatter(
                            hist, [r0 + iota16],
                            cur + tmphist[pl.ds(r0, 16)])
                    return 0

                lax.fori_loop(0, k_iter, hsum_g, 0)
                return 0

            lax.fori_loop(0, n_sub, hsum_t, 0)

            def div_body(k, _):
                gi = s + n_sub * k

                @pl.when(gi < ngr)
                def _():
                    r0 = pl.multiple_of(16 * gi, 16)
                    pltpu.sync_copy(acc.at[pl.ds(r0, 16)], sumbuf)
                    cvec = hist[pl.ds(r0, 16)]
                    inv = jnp.where(cvec > 0, 1.0 / cvec, 0.0)
                    for r in range(16):
                        ir = inv[r]
                        for j in range(d // 16):
                            resbuf[r, pl.ds(16 * j, 16)] = (
                                sumbuf[r, pl.ds(16 * j, 16)] * ir)
                    row_out = pl.multiple_of(bp + r0, 16)
                    pltpu.sync_copy(resbuf, out_hbm.at[pl.ds(row_out, 16)])
                return 0

            lax.fori_loop(0, k_iter, div_body, 0)
            plsc.subcore_barrier()

    return functools.partial(
        pl.kernel,
        out_type=jax.ShapeDtypeStruct((n_seg, d), jnp.float32),
        compiler_params=pltpu.CompilerParams(needs_layout_passes=False),
        mesh=plsc.VectorSubcoreMesh(core_axis_name="c", subcore_axis_name="s"),
        interpret=interpret,
        scratch_types=[
            pltpu.VMEM((rows_per_tile,), jnp.int32),
            pltpu.VMEM((cap,), jnp.int32),
            pltpu.VMEM((g, d), jnp.float32),
            pltpu.VMEM((g, d), jnp.float32),
            pltpu.VMEM((2, g), jnp.int32),
            pltpu.VMEM((g,), jnp.int32),
            pltpu.VMEM((g,), jnp.int32),
            pltpu.VMEM((histpad,), jnp.float32),
            pltpu.VMEM((histpad,), jnp.float32),
            pltpu.VMEM((16, d), jnp.float32),
            pltpu.VMEM((16, d), jnp.float32),
            pltpu.VMEM_SHARED((acc_rows, d), jnp.float32),
            pltpu.VMEM_SHARED((n_sub, histpad), jnp.float32),
            pltpu.SemaphoreType.DMA,
            pltpu.SemaphoreType.DMA,
            pltpu.SemaphoreType.DMA,
            pltpu.SemaphoreType.DMA,
        ],
    )(_body)


_mean_pool_sc = _make_kernel()


def kernel(fine_scale_h_d, prolongation_map_fine_to_coarse):
    return _mean_pool_sc(fine_scale_h_d, prolongation_map_fine_to_coarse)


# speedup vs baseline: 4.3516x; 1.0240x over previous
import functools

import jax
import jax.numpy as jnp
import numpy as np
from jax import lax
from jax.experimental import pallas as pl
from jax.experimental.pallas import tpu as pltpu
from jax.experimental.pallas import tpu_sc as plsc

N_ROWS = 320000
D = 128
N_SEG = 50000
N_PASS = 4
SEGCHUNK = 6256
G = 64
RID_BITS = 19


def _make_kernel(n_rows=N_ROWS, d=D, n_seg=N_SEG, n_pass=N_PASS,
                 segchunk=SEGCHUNK, g=G, rid_bits=RID_BITS, interpret=False):
    n_cores, n_sub = 2, 16
    n_regions = n_cores * n_pass
    last_len = n_seg - (n_regions - 1) * segchunk
    assert 0 < last_len <= segchunk and last_len % 16 == 0
    assert segchunk % 16 == 0 and n_rows % n_sub == 0
    rows_per_tile = n_rows // n_sub
    assert rows_per_tile % 16 == 0
    cap = ((rows_per_tile + g - 1) // g) * g + g
    trash = segchunk
    acc_rows = trash + 16
    ngroups = segchunk // 16
    ngroups_last = last_len // 16
    k_iter = (ngroups + n_sub - 1) // n_sub
    vec_iters = rows_per_tile // 16
    histpad = ((segchunk + 127) // 128) * 128
    hist_iters = histpad // 16
    rid_mask = (1 << rid_bits) - 1
    gsh = g.bit_length() - 1
    assert n_rows <= (1 << rid_bits) and (trash << rid_bits) < (1 << 32)
    trash_word = np.uint32(trash << rid_bits).astype(np.int32)

    def _body(fine_hbm, seg_hbm, out_hbm,
              segbuf, arena, rowbuf_a, rowbuf_b, loc_stage, rid_a, rid_b,
              hist, tmphist, sumbuf, resbuf, acc, hist_all,
              gsem_a, gsem_b, ssem_a, ssem_b, zsem):
        c = lax.axis_index("c")
        s = lax.axis_index("s")
        iota16 = lax.iota(jnp.int32, 16)
        fzero = jnp.zeros((16,), jnp.float32)
        fone = jnp.ones((16,), jnp.float32)
        trash_packed = jnp.full((16,), trash_word, jnp.int32)


        row_base = s * rows_per_tile
        pltpu.sync_copy(seg_hbm.at[pl.ds(row_base, rows_per_tile)], segbuf)

        ridbase = row_base + iota16

        for p in range(n_pass):
            q = c * n_pass + p
            bp = q * segchunk
            ngr = jnp.where(q == n_regions - 1, ngroups_last, ngroups)

            for r in range(16):
                for j in range(d // 16):
                    resbuf[r, pl.ds(16 * j, 16)] = fzero

            def zfire_body(k, _):
                gi = s + n_sub * k

                @pl.when(gi < ngr)
                def _():
                    r0 = pl.multiple_of(16 * gi, 16)
                    pltpu.async_copy(resbuf, acc.at[pl.ds(r0, 16)], zsem)
                return 0

            lax.fori_loop(0, k_iter, zfire_body, 0)

            def hzero_body(i, _):
                plsc.store_scatter(hist, [16 * i + iota16], fzero)
                return 0

            lax.fori_loop(0, hist_iters, hzero_body, 0)

            def comp_body(i, off):
                seg = segbuf[pl.ds(16 * i, 16)]
                loc = seg - bp
                m = (loc >= 0) & (loc < segchunk)
                locc = jnp.where(m, loc, 0)
                cum = plsc.cumsum(m.astype(jnp.int32))
                pos = jnp.where(m, off + cum - 1, cap - 16)
                packed = (ridbase + 16 * i) | (locc << rid_bits)
                plsc.store_scatter(arena, [pos], packed, mask=m)
                plsc.addupdate_scatter(hist, [locc], fone, mask=m)
                return off + jnp.max(cum)

            off = lax.fori_loop(0, vec_iters, comp_body, jnp.int32(0))

            pltpu.sync_copy(hist, hist_all.at[s])

            nch = lax.shift_right_logical(off + (g - 1), gsh)
            fl = jnp.bitwise_and(off, -16)
            keep = iota16 < (off - fl)
            plsc.store_scatter(arena, [fl + iota16], trash_packed,
                               mask=jnp.logical_not(keep))
            n_pad = jnp.maximum(
                lax.shift_right_arithmetic(nch * g - fl - 16, 4), 0)
            for k in range(g // 16 - 1):
                @pl.when(k < n_pad)
                def _():
                    plsc.store_scatter(arena, [fl + 16 + 16 * k + iota16],
                                       trash_packed)

            def zdrain_body(k, _):
                gi = s + n_sub * k

                @pl.when(gi < ngr)
                def _():
                    r0 = pl.multiple_of(16 * gi, 16)
                    pltpu.make_async_copy(resbuf, acc.at[pl.ds(r0, 16)],
                                          zsem).wait()
                return 0

            lax.fori_loop(0, k_iter, zdrain_body, 0)
            plsc.subcore_barrier()

            def unpack(gi, rid_stage, slot):
                gg = gi * g
                for j in range(g // 16):
                    v = arena[pl.ds(gg + 16 * j, 16)]
                    rid_stage[pl.ds(16 * j, 16)] = v & rid_mask
                    loc_stage[slot, pl.ds(16 * j, 16)] = (
                        lax.shift_right_logical(v, rid_bits))

            def acc_body(it, _):
                ga = 2 * it
                gb = 2 * it + 1
                b_ok = gb < nch
                unpack(ga, rid_a, 0)
                cp_ga = pltpu.async_copy(fine_hbm.at[rid_a], rowbuf_a,
                                         gsem_a)

                @pl.when(b_ok)
                def _():
                    unpack(gb, rid_b, 1)
                    pltpu.async_copy(fine_hbm.at[rid_b], rowbuf_b, gsem_b)

                cp_ga.wait()
                cp_sa = pltpu.async_copy(rowbuf_a, acc.at[loc_stage.at[0]],
                                         ssem_a, add=True)

                @pl.when(b_ok)
                def _():
                    pltpu.make_async_copy(fine_hbm.at[rid_b], rowbuf_b,
                                          gsem_b).wait()
                    pltpu.async_copy(rowbuf_b, acc.at[loc_stage.at[1]],
                                     ssem_b, add=True)

                cp_sa.wait()

                @pl.when(b_ok)
                def _():
                    pltpu.make_async_copy(rowbuf_b,
                                          acc.at[loc_stage.at[1]],
                                          ssem_b).wait()
                return 0

            n_it = lax.shift_right_logical(nch + 1, 1)
            lax.fori_loop(0, n_it, acc_body, 0)
            plsc.subcore_barrier()

            def hzero2_body(k, _):
                gi = s + n_sub * k

                @pl.when(gi < ngr)
                def _():
                    plsc.store_scatter(hist, [16 * gi + iota16], fzero)
                return 0

            lax.fori_loop(0, k_iter, hzero2_body, 0)

            def hsum_t(t, _):
                pltpu.sync_copy(hist_all.at[t], tmphist)

                def hsum_g(k, _):
                    gi = s + n_sub * k

                    @pl.when(gi < ngr)
                    def _():
                        r0 = 16 * gi
                        cur = hist[pl.ds(r0, 16)]
                        plsc.store_scatter(
                            hist, [r0 + iota16],
                            cur + tmphist[pl.ds(r0, 16)])
                    return 0

                lax.fori_loop(0, k_iter, hsum_g, 0)
                return 0

            lax.fori_loop(0, n_sub, hsum_t, 0)

            def div_body(k, _):
                gi = s + n_sub * k

                @pl.when(gi < ngr)
                def _():
                    r0 = pl.multiple_of(16 * gi, 16)
                    pltpu.sync_copy(acc.at[pl.ds(r0, 16)], sumbuf)
                    cvec = hist[pl.ds(r0, 16)]
                    inv = jnp.where(cvec > 0, 1.0 / cvec, 0.0)
                    for r in range(16):
                        ir = inv[r]
                        for j in range(d // 16):
                            resbuf[r, pl.ds(16 * j, 16)] = (
                                sumbuf[r, pl.ds(16 * j, 16)] * ir)
                    row_out = pl.multiple_of(bp + r0, 16)
                    pltpu.sync_copy(resbuf, out_hbm.at[pl.ds(row_out, 16)])
                return 0

            lax.fori_loop(0, k_iter, div_body, 0)
            plsc.subcore_barrier()

    return functools.partial(
        pl.kernel,
        out_type=jax.ShapeDtypeStruct((n_seg, d), jnp.float32),
        compiler_params=pltpu.CompilerParams(needs_layout_passes=False),
        mesh=plsc.VectorSubcoreMesh(core_axis_name="c", subcore_axis_name="s"),
        interpret=interpret,
        scratch_types=[
            pltpu.VMEM((rows_per_tile,), jnp.int32),
            pltpu.VMEM((cap,), jnp.int32),
            pltpu.VMEM((g, d), jnp.float32),
            pltpu.VMEM((g, d), jnp.float32),
            pltpu.VMEM((2, g), jnp.int32),
            pltpu.VMEM((g,), jnp.int32),
            pltpu.VMEM((g,), jnp.int32),
            pltpu.VMEM((histpad,), jnp.float32),
            pltpu.VMEM((histpad,), jnp.float32),
            pltpu.VMEM((16, d), jnp.float32),
            pltpu.VMEM((16, d), jnp.float32),
            pltpu.VMEM_SHARED((acc_rows, d), jnp.float32),
            pltpu.VMEM_SHARED((n_sub, histpad), jnp.float32),
            pltpu.SemaphoreType.DMA,
            pltpu.SemaphoreType.DMA,
            pltpu.SemaphoreType.DMA,
            pltpu.SemaphoreType.DMA,
            pltpu.SemaphoreType.DMA,
        ],
    )(_body)


_mean_pool_sc = _make_kernel()


def kernel(fine_scale_h_d, prolongation_map_fine_to_coarse):
    return _mean_pool_sc(fine_scale_h_d, prolongation_map_fine_to_coarse)


# speedup vs baseline: 4.4283x; 1.0176x over previous
import functools

import jax
import jax.numpy as jnp
import numpy as np
from jax import lax
from jax.experimental import pallas as pl
from jax.experimental.pallas import tpu as pltpu
from jax.experimental.pallas import tpu_sc as plsc

N_ROWS = 320000
D = 128
N_SEG = 50000
N_PASS = 4
SEGCHUNK = 6256
G = 64
RID_BITS = 19


def _make_kernel(n_rows=N_ROWS, d=D, n_seg=N_SEG, n_pass=N_PASS,
                 segchunk=SEGCHUNK, g=G, rid_bits=RID_BITS, interpret=False):
    n_cores, n_sub = 2, 16
    n_regions = n_cores * n_pass
    last_len = n_seg - (n_regions - 1) * segchunk
    assert 0 < last_len <= segchunk and last_len % 16 == 0
    assert segchunk % 16 == 0 and n_rows % n_sub == 0
    rows_per_tile = n_rows // n_sub
    assert rows_per_tile % 16 == 0
    cap = ((rows_per_tile + g - 1) // g) * g + g
    trash = segchunk
    acc_rows = trash + 16
    ngroups = segchunk // 16
    ngroups_last = last_len // 16
    k_iter = (ngroups + n_sub - 1) // n_sub
    vec_iters = rows_per_tile // 16
    histpad = ((segchunk + 127) // 128) * 128
    hist_iters = histpad // 16
    rid_mask = (1 << rid_bits) - 1
    gsh = g.bit_length() - 1
    assert n_rows <= (1 << rid_bits) and (trash << rid_bits) < (1 << 32)
    trash_word = np.uint32(trash << rid_bits).astype(np.int32)

    def _body(fine_hbm, seg_hbm, out_hbm,
              segbuf, arena, rowbuf_a, rowbuf_b, loc_stage, rid_a, rid_b,
              hist, tmphist, sumbuf, resbuf, acc, hist_all,
              gsem_a, gsem_b, ssem_a, ssem_b, zsem):
        c = lax.axis_index("c")
        s = lax.axis_index("s")
        iota16 = lax.iota(jnp.int32, 16)
        fzero = jnp.zeros((16,), jnp.float32)
        fone = jnp.ones((16,), jnp.float32)
        trash_packed = jnp.full((16,), trash_word, jnp.int32)


        row_base = s * rows_per_tile
        pltpu.sync_copy(seg_hbm.at[pl.ds(row_base, rows_per_tile)], segbuf)

        ridbase = row_base + iota16

        for p in range(n_pass):
            q = c * n_pass + p
            bp = q * segchunk
            ngr = jnp.where(q == n_regions - 1, ngroups_last, ngroups)

            for r in range(16):
                for j in range(d // 16):
                    resbuf[r, pl.ds(16 * j, 16)] = fzero

            def zfire_body(k, _):
                gi = s + n_sub * k

                @pl.when(gi < ngr)
                def _():
                    r0 = pl.multiple_of(16 * gi, 16)
                    pltpu.async_copy(resbuf, acc.at[pl.ds(r0, 16)], zsem)
                return 0

            lax.fori_loop(0, k_iter, zfire_body, 0)

            def hzero_body(i, _):
                plsc.store_scatter(hist, [16 * i + iota16], fzero)
                return 0

            lax.fori_loop(0, hist_iters, hzero_body, 0)

            def comp_body(i, off):
                seg = segbuf[pl.ds(16 * i, 16)]
                loc = seg - bp
                m = (loc >= 0) & (loc < segchunk)
                locc = jnp.where(m, loc, 0)
                cum = plsc.cumsum(m.astype(jnp.int32))
                pos = jnp.where(m, off + cum - 1, cap - 16)
                packed = (ridbase + 16 * i) | (locc << rid_bits)
                plsc.store_scatter(arena, [pos], packed, mask=m)
                plsc.addupdate_scatter(hist, [locc], fone, mask=m)
                n = plsc.all_reduce_population_count(m)[0]
                return off + n

            off = lax.fori_loop(0, vec_iters, comp_body, jnp.int32(0))

            pltpu.sync_copy(hist, hist_all.at[s])

            nch = lax.shift_right_logical(off + (g - 1), gsh)
            fl = jnp.bitwise_and(off, -16)
            keep = iota16 < (off - fl)
            plsc.store_scatter(arena, [fl + iota16], trash_packed,
                               mask=jnp.logical_not(keep))
            n_pad = jnp.maximum(
                lax.shift_right_arithmetic(nch * g - fl - 16, 4), 0)
            for k in range(g // 16 - 1):
                @pl.when(k < n_pad)
                def _():
                    plsc.store_scatter(arena, [fl + 16 + 16 * k + iota16],
                                       trash_packed)

            def zdrain_body(k, _):
                gi = s + n_sub * k

                @pl.when(gi < ngr)
                def _():
                    r0 = pl.multiple_of(16 * gi, 16)
                    pltpu.make_async_copy(resbuf, acc.at[pl.ds(r0, 16)],
                                          zsem).wait()
                return 0

            lax.fori_loop(0, k_iter, zdrain_body, 0)
            plsc.subcore_barrier()

            def unpack(gi, rid_stage, slot):
                gg = gi * g
                for j in range(g // 16):
                    v = arena[pl.ds(gg + 16 * j, 16)]
                    rid_stage[pl.ds(16 * j, 16)] = v & rid_mask
                    loc_stage[slot, pl.ds(16 * j, 16)] = (
                        lax.shift_right_logical(v, rid_bits))

            def acc_body(it, _):
                ga = 2 * it
                gb = 2 * it + 1
                b_ok = gb < nch
                unpack(ga, rid_a, 0)
                cp_ga = pltpu.async_copy(fine_hbm.at[rid_a], rowbuf_a,
                                         gsem_a)

                @pl.when(b_ok)
                def _():
                    unpack(gb, rid_b, 1)
                    pltpu.async_copy(fine_hbm.at[rid_b], rowbuf_b, gsem_b)

                cp_ga.wait()
                cp_sa = pltpu.async_copy(rowbuf_a, acc.at[loc_stage.at[0]],
                                         ssem_a, add=True)

                @pl.when(b_ok)
                def _():
                    pltpu.make_async_copy(fine_hbm.at[rid_b], rowbuf_b,
                                          gsem_b).wait()
                    pltpu.async_copy(rowbuf_b, acc.at[loc_stage.at[1]],
                                     ssem_b, add=True)

                cp_sa.wait()

                @pl.when(b_ok)
                def _():
                    pltpu.make_async_copy(rowbuf_b,
                                          acc.at[loc_stage.at[1]],
                                          ssem_b).wait()
                return 0

            n_it = lax.shift_right_logical(nch + 1, 1)
            lax.fori_loop(0, n_it, acc_body, 0)
            plsc.subcore_barrier()

            def hzero2_body(k, _):
                gi = s + n_sub * k

                @pl.when(gi < ngr)
                def _():
                    plsc.store_scatter(hist, [16 * gi + iota16], fzero)
                return 0

            lax.fori_loop(0, k_iter, hzero2_body, 0)

            def hsum_t(t, _):
                pltpu.sync_copy(hist_all.at[t], tmphist)

                def hsum_g(k, _):
                    gi = s + n_sub * k

                    @pl.when(gi < ngr)
                    def _():
                        r0 = 16 * gi
                        cur = hist[pl.ds(r0, 16)]
                        plsc.store_scatter(
                            hist, [r0 + iota16],
                            cur + tmphist[pl.ds(r0, 16)])
                    return 0

                lax.fori_loop(0, k_iter, hsum_g, 0)
                return 0

            lax.fori_loop(0, n_sub, hsum_t, 0)

            def div_body(k, _):
                gi = s + n_sub * k

                @pl.when(gi < ngr)
                def _():
                    r0 = pl.multiple_of(16 * gi, 16)
                    pltpu.sync_copy(acc.at[pl.ds(r0, 16)], sumbuf)
                    cvec = hist[pl.ds(r0, 16)]
                    inv = jnp.where(cvec > 0, 1.0 / cvec, 0.0)
                    for r in range(16):
                        ir = inv[r]
                        for j in range(d // 16):
                            resbuf[r, pl.ds(16 * j, 16)] = (
                                sumbuf[r, pl.ds(16 * j, 16)] * ir)
                    row_out = pl.multiple_of(bp + r0, 16)
                    pltpu.sync_copy(resbuf, out_hbm.at[pl.ds(row_out, 16)])
                return 0

            lax.fori_loop(0, k_iter, div_body, 0)
            plsc.subcore_barrier()

    return functools.partial(
        pl.kernel,
        out_type=jax.ShapeDtypeStruct((n_seg, d), jnp.float32),
        compiler_params=pltpu.CompilerParams(needs_layout_passes=False),
        mesh=plsc.VectorSubcoreMesh(core_axis_name="c", subcore_axis_name="s"),
        interpret=interpret,
        scratch_types=[
            pltpu.VMEM((rows_per_tile,), jnp.int32),
            pltpu.VMEM((cap,), jnp.int32),
            pltpu.VMEM((g, d), jnp.float32),
            pltpu.VMEM((g, d), jnp.float32),
            pltpu.VMEM((2, g), jnp.int32),
            pltpu.VMEM((g,), jnp.int32),
            pltpu.VMEM((g,), jnp.int32),
            pltpu.VMEM((histpad,), jnp.float32),
            pltpu.VMEM((histpad,), jnp.float32),
            pltpu.VMEM((16, d), jnp.float32),
            pltpu.VMEM((16, d), jnp.float32),
            pltpu.VMEM_SHARED((acc_rows, d), jnp.float32),
            pltpu.VMEM_SHARED((n_sub, histpad), jnp.float32),
            pltpu.SemaphoreType.DMA,
            pltpu.SemaphoreType.DMA,
            pltpu.SemaphoreType.DMA,
            pltpu.SemaphoreType.DMA,
            pltpu.SemaphoreType.DMA,
        ],
    )(_body)


_mean_pool_sc = _make_kernel()


def kernel(fine_scale_h_d, prolongation_map_fine_to_coarse):
    return _mean_pool_sc(fine_scale_h_d, prolongation_map_fine_to_coarse)


# speedup vs baseline: 4.6100x; 1.0410x over previous
import functools

import jax
import jax.numpy as jnp
import numpy as np
from jax import lax
from jax.experimental import pallas as pl
from jax.experimental.pallas import tpu as pltpu
from jax.experimental.pallas import tpu_sc as plsc

N_ROWS = 320000
D = 128
N_SEG = 50000
N_PASS = 4
SEGCHUNK = 6256
G = 64
RID_BITS = 19


def _make_kernel(n_rows=N_ROWS, d=D, n_seg=N_SEG, n_pass=N_PASS,
                 segchunk=SEGCHUNK, g=G, rid_bits=RID_BITS, interpret=False):
    n_cores, n_sub = 2, 16
    n_regions = n_cores * n_pass
    last_len = n_seg - (n_regions - 1) * segchunk
    assert 0 < last_len <= segchunk and last_len % 16 == 0
    assert segchunk % 16 == 0 and n_rows % n_sub == 0
    rows_per_tile = n_rows // n_sub
    assert rows_per_tile % 16 == 0
    cap = ((rows_per_tile + g - 1) // g) * g + g
    trash = segchunk
    acc_rows = trash + 16
    ngroups = segchunk // 16
    ngroups_last = last_len // 16
    k_iter = (ngroups + n_sub - 1) // n_sub
    vec_iters = rows_per_tile // 16
    histpad = ((segchunk + 127) // 128) * 128
    hist_iters = histpad // 16
    rid_mask = (1 << rid_bits) - 1
    gsh = g.bit_length() - 1
    assert n_rows <= (1 << rid_bits) and (trash << rid_bits) < (1 << 32)
    trash_word = np.uint32(trash << rid_bits).astype(np.int32)

    def _body(fine_hbm, seg_hbm, out_hbm,
              segbuf, arena, rowbuf_a, rowbuf_b, loc_stage, rid_a, rid_b,
              hist, tmphist, sumbuf, resbuf, acc, hist_all,
              gsem_a, gsem_b, ssem_a, ssem_b, zsem):
        c = lax.axis_index("c")
        s = lax.axis_index("s")
        iota16 = lax.iota(jnp.int32, 16)
        fzero = jnp.zeros((16,), jnp.float32)
        fone = jnp.ones((16,), jnp.float32)
        trash_packed = jnp.full((16,), trash_word, jnp.int32)


        row_base = s * rows_per_tile
        pltpu.sync_copy(seg_hbm.at[pl.ds(row_base, rows_per_tile)], segbuf)

        ridbase = row_base + iota16

        for p in range(n_pass):
            q = c * n_pass + p
            bp = q * segchunk
            ngr = jnp.where(q == n_regions - 1, ngroups_last, ngroups)
            gbase = lax.shift_right_logical(ngr, 4)
            grem = ngr - (gbase << 4)
            glo = gbase * s + jnp.minimum(s, grem)
            gcnt = gbase + jnp.where(s < grem, 1, 0)

            for r in range(16):
                for j in range(d // 16):
                    resbuf[r, pl.ds(16 * j, 16)] = fzero

            def zfire_body(k, _):
                gi = glo + k

                @pl.when(k < gcnt)
                def _():
                    r0 = pl.multiple_of(16 * gi, 16)
                    pltpu.async_copy(resbuf, acc.at[pl.ds(r0, 16)], zsem)
                return 0

            lax.fori_loop(0, k_iter, zfire_body, 0)

            def hzero_body(i, _):
                plsc.store_scatter(hist, [16 * i + iota16], fzero)
                return 0

            lax.fori_loop(0, hist_iters, hzero_body, 0)

            def comp_body(i, off):
                seg = segbuf[pl.ds(16 * i, 16)]
                loc = seg - bp
                m = (loc >= 0) & (loc < segchunk)
                locc = jnp.where(m, loc, 0)
                cum = plsc.cumsum(m.astype(jnp.int32))
                pos = jnp.where(m, off + cum - 1, cap - 16)
                packed = (ridbase + 16 * i) | (locc << rid_bits)
                plsc.store_scatter(arena, [pos], packed, mask=m)
                plsc.addupdate_scatter(hist, [locc], fone, mask=m)
                n = plsc.all_reduce_population_count(m)[0]
                return off + n

            off = lax.fori_loop(0, vec_iters, comp_body, jnp.int32(0))

            pltpu.sync_copy(hist, hist_all.at[pl.ds(pl.multiple_of(s * histpad, 128), histpad)])

            nch = lax.shift_right_logical(off + (g - 1), gsh)
            fl = jnp.bitwise_and(off, -16)
            keep = iota16 < (off - fl)
            plsc.store_scatter(arena, [fl + iota16], trash_packed,
                               mask=jnp.logical_not(keep))
            n_pad = jnp.maximum(
                lax.shift_right_arithmetic(nch * g - fl - 16, 4), 0)
            for k in range(g // 16 - 1):
                @pl.when(k < n_pad)
                def _():
                    plsc.store_scatter(arena, [fl + 16 + 16 * k + iota16],
                                       trash_packed)

            def zdrain_body(k, _):
                gi = glo + k

                @pl.when(k < gcnt)
                def _():
                    r0 = pl.multiple_of(16 * gi, 16)
                    pltpu.make_async_copy(resbuf, acc.at[pl.ds(r0, 16)],
                                          zsem).wait()
                return 0

            lax.fori_loop(0, k_iter, zdrain_body, 0)
            plsc.subcore_barrier()

            def unpack(gi, rid_stage, slot):
                gg = gi * g
                for j in range(g // 16):
                    v = arena[pl.ds(gg + 16 * j, 16)]
                    rid_stage[pl.ds(16 * j, 16)] = v & rid_mask
                    loc_stage[slot, pl.ds(16 * j, 16)] = (
                        lax.shift_right_logical(v, rid_bits))

            def acc_body(it, _):
                ga = 2 * it
                gb = 2 * it + 1
                b_ok = gb < nch
                unpack(ga, rid_a, 0)
                cp_ga = pltpu.async_copy(fine_hbm.at[rid_a], rowbuf_a,
                                         gsem_a)

                @pl.when(b_ok)
                def _():
                    unpack(gb, rid_b, 1)
                    pltpu.async_copy(fine_hbm.at[rid_b], rowbuf_b, gsem_b)

                cp_ga.wait()
                cp_sa = pltpu.async_copy(rowbuf_a, acc.at[loc_stage.at[0]],
                                         ssem_a, add=True)

                @pl.when(b_ok)
                def _():
                    pltpu.make_async_copy(fine_hbm.at[rid_b], rowbuf_b,
                                          gsem_b).wait()
                    pltpu.async_copy(rowbuf_b, acc.at[loc_stage.at[1]],
                                     ssem_b, add=True)

                cp_sa.wait()

                @pl.when(b_ok)
                def _():
                    pltpu.make_async_copy(rowbuf_b,
                                          acc.at[loc_stage.at[1]],
                                          ssem_b).wait()
                return 0

            n_it = lax.shift_right_logical(nch + 1, 1)
            lax.fori_loop(0, n_it, acc_body, 0)
            plsc.subcore_barrier()

            def hzero2_body(k, _):
                gi = glo + k

                @pl.when(k < gcnt)
                def _():
                    plsc.store_scatter(hist, [16 * gi + iota16], fzero)
                return 0

            lax.fori_loop(0, k_iter, hzero2_body, 0)

            span = 16 * k_iter
            h0 = pl.multiple_of(16 * glo, 16)

            def hsum_t(t, _):
                src_off = pl.multiple_of(t * histpad + h0, 16)
                pltpu.sync_copy(hist_all.at[pl.ds(src_off, span)], tmphist)

                def hsum_g(k, _):
                    gi = glo + k

                    @pl.when(k < gcnt)
                    def _():
                        r0 = 16 * gi
                        cur = hist[pl.ds(r0, 16)]
                        plsc.store_scatter(
                            hist, [r0 + iota16],
                            cur + tmphist[pl.ds(16 * k, 16)])
                    return 0

                lax.fori_loop(0, k_iter, hsum_g, 0)
                return 0

            lax.fori_loop(0, n_sub, hsum_t, 0)

            def div_body(k, _):
                gi = glo + k

                @pl.when(k < gcnt)
                def _():
                    r0 = pl.multiple_of(16 * gi, 16)
                    pltpu.sync_copy(acc.at[pl.ds(r0, 16)], sumbuf)
                    cvec = hist[pl.ds(r0, 16)]
                    inv = jnp.where(cvec > 0, 1.0 / cvec, 0.0)
                    for r in range(16):
                        ir = inv[r]
                        for j in range(d // 16):
                            resbuf[r, pl.ds(16 * j, 16)] = (
                                sumbuf[r, pl.ds(16 * j, 16)] * ir)
                    row_out = pl.multiple_of(bp + r0, 16)
                    pltpu.sync_copy(resbuf, out_hbm.at[pl.ds(row_out, 16)])
                return 0

            lax.fori_loop(0, k_iter, div_body, 0)
            plsc.subcore_barrier()

    return functools.partial(
        pl.kernel,
        out_type=jax.ShapeDtypeStruct((n_seg, d), jnp.float32),
        compiler_params=pltpu.CompilerParams(needs_layout_passes=False),
        mesh=plsc.VectorSubcoreMesh(core_axis_name="c", subcore_axis_name="s"),
        interpret=interpret,
        scratch_types=[
            pltpu.VMEM((rows_per_tile,), jnp.int32),
            pltpu.VMEM((cap,), jnp.int32),
            pltpu.VMEM((g, d), jnp.float32),
            pltpu.VMEM((g, d), jnp.float32),
            pltpu.VMEM((2, g), jnp.int32),
            pltpu.VMEM((g,), jnp.int32),
            pltpu.VMEM((g,), jnp.int32),
            pltpu.VMEM((histpad,), jnp.float32),
            pltpu.VMEM((16 * k_iter,), jnp.float32),
            pltpu.VMEM((16, d), jnp.float32),
            pltpu.VMEM((16, d), jnp.float32),
            pltpu.VMEM_SHARED((acc_rows, d), jnp.float32),
            pltpu.VMEM_SHARED((n_sub * histpad,), jnp.float32),
            pltpu.SemaphoreType.DMA,
            pltpu.SemaphoreType.DMA,
            pltpu.SemaphoreType.DMA,
            pltpu.SemaphoreType.DMA,
            pltpu.SemaphoreType.DMA,
        ],
    )(_body)


_mean_pool_sc = _make_kernel()


def kernel(fine_scale_h_d, prolongation_map_fine_to_coarse):
    return _mean_pool_sc(fine_scale_h_d, prolongation_map_fine_to_coarse)
